# baseline (speedup 1.0000x reference)
import functools

import jax
import jax.numpy as jnp
from jax import lax
from jax.experimental import pallas as pl
from jax.experimental.pallas import tpu as pltpu
from jax.experimental.pallas import tpu_sc as plsc

VOCAB = 49408
EMBED = 768
MAX_POS = 77
BATCH = 4096
SEQ = 77

NUM_WORKERS = 32
ROWS = BATCH * SEQ
ROWS_PER_W = ROWS // NUM_WORKERS
SEQS_PER_W = ROWS_PER_W // SEQ
CHUNK = 11
CHUNKS_PER_SEQ = SEQ // CHUNK
CHUNKS_PER_W = ROWS_PER_W // CHUNK
LANES = 16
VREGS_PER_ROW = EMBED // LANES


def _body(ids_hbm, tok_hbm, pos_hbm, out_hbm,
          idx_v, pos_v, rows_a, rows_b, sem_a, sem_b, osem_a, osem_b):
    wid = lax.axis_index("s") * 2 + lax.axis_index("c")
    base_row = wid * ROWS_PER_W

    pltpu.sync_copy(ids_hbm.at[wid], idx_v)
    pltpu.sync_copy(pos_hbm, pos_v)

    bufs = ((rows_a, sem_a, osem_a), (rows_b, sem_b, osem_b))

    def start_gather(c, buf, gsem):
        pltpu.async_copy(tok_hbm.at[idx_v.at[c]], buf, gsem)

    def add_pos_and_store(c, buf, osem):
        cb = lax.rem(c, CHUNKS_PER_SEQ)
        p0 = cb * CHUNK
        row0 = base_row + (c - cb) * CHUNK + p0

        def row_add(j, _):
            for v in range(VREGS_PER_ROW):
                sl = pl.ds(v * LANES, LANES)
                plsc.addupdate(buf.at[j, sl], pos_v[p0 + j, sl])
            return 0

        lax.fori_loop(0, CHUNK, row_add, 0, unroll=False)
        pltpu.async_copy(buf, out_hbm.at[pl.ds(row0, CHUNK)], osem)

    start_gather(0, bufs[0][0], bufs[0][1])
    start_gather(1, bufs[1][0], bufs[1][1])

    def loop_body(c2, _):
        for k in range(2):
            c = 2 * c2 + k
            buf, gsem, osem = bufs[k]
            pltpu.make_async_copy(tok_hbm.at[idx_v.at[c]], buf, gsem).wait()
            add_pos_and_store(c, buf, osem)
            pltpu.make_async_copy(
                buf, out_hbm.at[pl.ds(base_row, CHUNK)], osem
            ).wait()
            next_c = c + 2

            @pl.when(next_c < CHUNKS_PER_W)
            def _():
                start_gather(next_c, buf, gsem)

        return 0

    lax.fori_loop(0, CHUNKS_PER_W // 2, loop_body, 0, unroll=False)


@jax.jit
def _run(ids_grouped, token_embedding, position_embedding):
    mesh = plsc.VectorSubcoreMesh(core_axis_name="c", subcore_axis_name="s")
    out = pl.kernel(
        _body,
        out_type=jax.ShapeDtypeStruct((ROWS, EMBED), jnp.float32),
        mesh=mesh,
        compiler_params=pltpu.CompilerParams(use_tc_tiling_on_sc=False),
        scratch_types=[
            pltpu.VMEM((CHUNKS_PER_W, CHUNK), jnp.int32),
            pltpu.VMEM((MAX_POS, EMBED), jnp.float32),
            pltpu.VMEM((CHUNK, EMBED), jnp.float32),
            pltpu.VMEM((CHUNK, EMBED), jnp.float32),
            pltpu.SemaphoreType.DMA,
            pltpu.SemaphoreType.DMA,
            pltpu.SemaphoreType.DMA,
            pltpu.SemaphoreType.DMA,
        ],
    )(ids_grouped, token_embedding, position_embedding)
    return out


def kernel(input_ids, token_embedding, position_embedding):
    ids_grouped = input_ids.reshape(NUM_WORKERS, CHUNKS_PER_W, CHUNK)
    out = _run(ids_grouped, token_embedding, position_embedding)
    return out.reshape(BATCH, SEQ, EMBED)


# speedup vs baseline: 1.4631x; 1.4631x over previous
import functools

import jax
import jax.numpy as jnp
from jax import lax
from jax.experimental import pallas as pl
from jax.experimental.pallas import tpu as pltpu
from jax.experimental.pallas import tpu_sc as plsc

VOCAB = 49408
EMBED = 768
MAX_POS = 77
BATCH = 4096
SEQ = 77

NUM_WORKERS = 32
ROWS = BATCH * SEQ
ROWS_PER_W = ROWS // NUM_WORKERS
SEQS_PER_W = ROWS_PER_W // SEQ
CHUNK = 11
CHUNKS_PER_SEQ = SEQ // CHUNK
CHUNKS_PER_W = ROWS_PER_W // CHUNK
LANES = 16
VREGS_PER_ROW = EMBED // LANES


def _body(ids_hbm, tok_hbm, pos_hbm, out_hbm,
          idx_v, pos_v, rows_a, rows_b, sem_a, sem_b, osem_a, osem_b):
    wid = lax.axis_index("s") * 2 + lax.axis_index("c")
    base_row = wid * ROWS_PER_W

    pltpu.sync_copy(ids_hbm.at[wid], idx_v)
    pltpu.sync_copy(pos_hbm, pos_v)

    bufs = ((rows_a, sem_a, osem_a), (rows_b, sem_b, osem_b))

    def start_gather(c, buf, gsem):
        pltpu.async_copy(tok_hbm.at[idx_v.at[c]], buf, gsem)

    GROUP = 12

    def add_pos_and_store(c, buf, osem, cb):
        p0 = cb * CHUNK
        row0 = base_row + c * CHUNK

        def row_add(j):
            for g in range(0, VREGS_PER_ROW, GROUP):
                vals = [pos_v[p0 + j, pl.ds(v * LANES, LANES)]
                        for v in range(g, g + GROUP)]
                for i, v in enumerate(range(g, g + GROUP)):
                    plsc.addupdate(buf.at[j, pl.ds(v * LANES, LANES)], vals[i])

        plsc.parallel_loop(0, CHUNK, unroll=2)(row_add)
        pltpu.async_copy(buf, out_hbm.at[pl.ds(row0, CHUNK)], osem)

    start_gather(0, bufs[0][0], bufs[0][1])
    start_gather(1, bufs[1][0], bufs[1][1])

    def loop_body(c2, cb):
        for k in range(2):
            c = 2 * c2 + k
            buf, gsem, osem = bufs[k]
            pltpu.make_async_copy(tok_hbm.at[idx_v.at[c]], buf, gsem).wait()
            add_pos_and_store(c, buf, osem, cb)
            cb = lax.select(cb == CHUNKS_PER_SEQ - 1, 0, cb + 1)
            pltpu.make_async_copy(
                buf, out_hbm.at[pl.ds(base_row, CHUNK)], osem
            ).wait()
            next_c = c + 2

            @pl.when(next_c < CHUNKS_PER_W)
            def _():
                start_gather(next_c, buf, gsem)

        return cb

    lax.fori_loop(0, CHUNKS_PER_W // 2, loop_body, jnp.int32(0), unroll=False)


@jax.jit
def _run(ids_grouped, token_embedding, position_embedding):
    mesh = plsc.VectorSubcoreMesh(core_axis_name="c", subcore_axis_name="s")
    out = pl.kernel(
        _body,
        out_type=jax.ShapeDtypeStruct((ROWS, EMBED), jnp.float32),
        mesh=mesh,
        compiler_params=pltpu.CompilerParams(use_tc_tiling_on_sc=False),
        scratch_types=[
            pltpu.VMEM((CHUNKS_PER_W, CHUNK), jnp.int32),
            pltpu.VMEM((MAX_POS, EMBED), jnp.float32),
            pltpu.VMEM((CHUNK, EMBED), jnp.float32),
            pltpu.VMEM((CHUNK, EMBED), jnp.float32),
            pltpu.SemaphoreType.DMA,
            pltpu.SemaphoreType.DMA,
            pltpu.SemaphoreType.DMA,
            pltpu.SemaphoreType.DMA,
        ],
    )(ids_grouped, token_embedding, position_embedding)
    return out


def kernel(input_ids, token_embedding, position_embedding):
    ids_grouped = input_ids.reshape(NUM_WORKERS, CHUNKS_PER_W, CHUNK)
    out = _run(ids_grouped, token_embedding, position_embedding)
    return out.reshape(BATCH, SEQ, EMBED)


# speedup vs baseline: 1.5250x; 1.0424x over previous
import jax
import jax.numpy as jnp
from jax import lax
from jax.experimental import pallas as pl
from jax.experimental.pallas import tpu as pltpu
from jax.experimental.pallas import tpu_sc as plsc

VOCAB = 49408
EMBED = 768
MAX_POS = 77
BATCH = 4096
SEQ = 77

NUM_WORKERS = 32
ROWS = BATCH * SEQ
ROWS_PER_W = ROWS // NUM_WORKERS
SEQS_PER_W = ROWS_PER_W // SEQ
CHUNK = 11
CHUNKS_PER_SEQ = SEQ // CHUNK
CHUNKS_PER_W = ROWS_PER_W // CHUNK
LANES = 16
VREGS_PER_ROW = EMBED // LANES
NBUF = 4


def _body(ids_hbm, tok_hbm, pos_hbm, out_hbm, idx_v, pos_v,
          buf0, buf1, buf2, buf3, g0, g1, g2, g3, o0, o1, o2, o3):
    wid = lax.axis_index("s") * 2 + lax.axis_index("c")
    seq0 = wid * SEQS_PER_W

    pltpu.sync_copy(ids_hbm.at[wid], idx_v)
    pltpu.sync_copy(pos_hbm, pos_v)

    bufs = ((buf0, g0, o0), (buf1, g1, o1), (buf2, g2, o2), (buf3, g3, o3))

    def start_gather(c, buf, gsem):
        pltpu.async_copy(tok_hbm.at[idx_v.at[c]], buf, gsem)

    def add_pos(buf, p0):
        def row_add(j, carry):
            prow = p0 + j

            def one_vreg(v):
                sl = pl.ds(v * LANES, LANES)
                plsc.addupdate(buf.at[j, sl], pos_v[prow, sl])

            plsc.parallel_loop(0, VREGS_PER_ROW, unroll=8)(one_vreg)
            return carry

        lax.fori_loop(0, CHUNK, row_add, 0, unroll=False)

    start_gather(0, bufs[0][0], bufs[0][1])
    start_gather(1, bufs[1][0], bufs[1][1])

    def loop_body(c2, carry):
        b, cb = carry
        for k in range(NBUF):
            c = NBUF * c2 + k
            buf, gsem, osem = bufs[k]
            pltpu.make_async_copy(tok_hbm.at[idx_v.at[c]], buf, gsem).wait()
            p0 = cb * CHUNK
            add_pos(buf, p0)
            pltpu.async_copy(buf, out_hbm.at[seq0 + b, pl.ds(p0, CHUNK)], osem)
            wrap = cb == CHUNKS_PER_SEQ - 1
            b = b + wrap.astype(jnp.int32)
            cb = lax.select(wrap, jnp.int32(0), cb + 1)

            nbuf_, ngsem, nosem = bufs[(k + 2) % NBUF]

            @pl.when(c + 2 < CHUNKS_PER_W)
            def _():
                def drain_and_fetch():
                    pltpu.make_async_copy(
                        nbuf_, out_hbm.at[0, pl.ds(0, CHUNK)], nosem
                    ).wait()
                    start_gather(c + 2, nbuf_, ngsem)

                if k < 2:
                    @pl.when(c2 > 0)
                    def _():
                        drain_and_fetch()

                    @pl.when(c2 == 0)
                    def _():
                        start_gather(c + 2, nbuf_, ngsem)
                else:
                    drain_and_fetch()

        return (b, cb)

    lax.fori_loop(0, CHUNKS_PER_W // NBUF, loop_body,
                  (jnp.int32(0), jnp.int32(0)), unroll=False)

    for k in range(NBUF):
        buf, _, osem = bufs[k]
        pltpu.make_async_copy(
            buf, out_hbm.at[0, pl.ds(0, CHUNK)], osem
        ).wait()


@jax.jit
def _run(ids_grouped, token_embedding, position_embedding):
    mesh = plsc.VectorSubcoreMesh(core_axis_name="c", subcore_axis_name="s")
    out = pl.kernel(
        _body,
        out_type=jax.ShapeDtypeStruct((BATCH, SEQ, EMBED), jnp.float32),
        mesh=mesh,
        compiler_params=pltpu.CompilerParams(use_tc_tiling_on_sc=False),
        scratch_types=[
            pltpu.VMEM((CHUNKS_PER_W, CHUNK), jnp.int32),
            pltpu.VMEM((MAX_POS, EMBED), jnp.float32),
            pltpu.VMEM((CHUNK, EMBED), jnp.float32),
            pltpu.VMEM((CHUNK, EMBED), jnp.float32),
            pltpu.VMEM((CHUNK, EMBED), jnp.float32),
            pltpu.VMEM((CHUNK, EMBED), jnp.float32),
            pltpu.SemaphoreType.DMA,
            pltpu.SemaphoreType.DMA,
            pltpu.SemaphoreType.DMA,
            pltpu.SemaphoreType.DMA,
            pltpu.SemaphoreType.DMA,
            pltpu.SemaphoreType.DMA,
            pltpu.SemaphoreType.DMA,
            pltpu.SemaphoreType.DMA,
        ],
    )(ids_grouped, token_embedding, position_embedding)
    return out


def kernel(input_ids, token_embedding, position_embedding):
    ids_grouped = input_ids.reshape(NUM_WORKERS, CHUNKS_PER_W, CHUNK)
    return _run(ids_grouped, token_embedding, position_embedding)


# speedup vs baseline: 1.5260x; 1.0007x over previous
import functools

import jax
import jax.numpy as jnp
from jax import lax
from jax.experimental import layout as jex_layout
from jax.experimental import pallas as pl
from jax.experimental.pallas import tpu as pltpu
from jax.experimental.pallas import tpu_sc as plsc

VOCAB = 49408
EMBED = 768
MAX_POS = 77
BATCH = 4096
SEQ = 77

NUM_WORKERS = 32
ROWS = BATCH * SEQ
ROWS_PER_W = ROWS // NUM_WORKERS
SEQS_PER_W = ROWS_PER_W // SEQ
CHUNK = 11
CHUNKS_PER_SEQ = SEQ // CHUNK
CHUNKS_PER_W = ROWS_PER_W // CHUNK
LANES = 16
VREGS_PER_ROW = EMBED // LANES
NBUF = 4


def _body(ids_hbm, tok_hbm, pos_hbm, out_hbm, idx_v, pos_v,
          buf0, buf1, buf2, buf3, g0, g1, g2, g3, o0, o1, o2, o3):
    wid = lax.axis_index("s") * 2 + lax.axis_index("c")
    seq0 = wid * SEQS_PER_W

    pltpu.sync_copy(ids_hbm.at[wid], idx_v)
    pltpu.sync_copy(pos_hbm, pos_v)

    bufs = ((buf0, g0, o0), (buf1, g1, o1), (buf2, g2, o2), (buf3, g3, o3))

    def start_gather(c, buf, gsem):
        pltpu.async_copy(tok_hbm.at[idx_v.at[c]], buf, gsem)

    def add_pos(buf, p0):
        def row_add(j, carry):
            prow = p0 + j

            def one_vreg(v):
                sl = pl.ds(v * LANES, LANES)
                plsc.addupdate(buf.at[j, sl], pos_v[prow, sl])

            plsc.parallel_loop(0, VREGS_PER_ROW, unroll=8)(one_vreg)
            return carry

        lax.fori_loop(0, CHUNK, row_add, 0, unroll=False)

    start_gather(0, bufs[0][0], bufs[0][1])
    start_gather(1, bufs[1][0], bufs[1][1])

    def loop_body(c2, carry):
        b, cb = carry
        for k in range(NBUF):
            c = NBUF * c2 + k
            buf, gsem, osem = bufs[k]
            pltpu.make_async_copy(tok_hbm.at[idx_v.at[c]], buf, gsem).wait()
            p0 = cb * CHUNK
            add_pos(buf, p0)
            pltpu.async_copy(buf, out_hbm.at[seq0 + b, pl.ds(p0, CHUNK)], osem)
            wrap = cb == CHUNKS_PER_SEQ - 1
            b = b + wrap.astype(jnp.int32)
            cb = lax.select(wrap, jnp.int32(0), cb + 1)

            nbuf_, ngsem, nosem = bufs[(k + 2) % NBUF]

            @pl.when(c + 2 < CHUNKS_PER_W)
            def _():
                def drain_and_fetch():
                    pltpu.make_async_copy(
                        nbuf_, out_hbm.at[0, pl.ds(0, CHUNK)], nosem
                    ).wait()
                    start_gather(c + 2, nbuf_, ngsem)

                if k < 2:
                    @pl.when(c2 > 0)
                    def _():
                        drain_and_fetch()

                    @pl.when(c2 == 0)
                    def _():
                        start_gather(c + 2, nbuf_, ngsem)
                else:
                    drain_and_fetch()

        return (b, cb)

    lax.fori_loop(0, CHUNKS_PER_W // NBUF, loop_body,
                  (jnp.int32(0), jnp.int32(0)), unroll=False)

    for k in range(NBUF):
        buf, _, osem = bufs[k]
        pltpu.make_async_copy(
            buf, out_hbm.at[0, pl.ds(0, CHUNK)], osem
        ).wait()


def _run(ids_grouped, token_embedding, position_embedding):
    mesh = plsc.VectorSubcoreMesh(core_axis_name="c", subcore_axis_name="s")
    out = pl.kernel(
        _body,
        out_type=jax.ShapeDtypeStruct((BATCH, SEQ, EMBED), jnp.float32),
        mesh=mesh,
        compiler_params=pltpu.CompilerParams(use_tc_tiling_on_sc=False),
        scratch_types=[
            pltpu.VMEM((CHUNKS_PER_W, CHUNK), jnp.int32),
            pltpu.VMEM((MAX_POS, EMBED), jnp.float32),
            pltpu.VMEM((CHUNK, EMBED), jnp.float32),
            pltpu.VMEM((CHUNK, EMBED), jnp.float32),
            pltpu.VMEM((CHUNK, EMBED), jnp.float32),
            pltpu.VMEM((CHUNK, EMBED), jnp.float32),
            pltpu.SemaphoreType.DMA,
            pltpu.SemaphoreType.DMA,
            pltpu.SemaphoreType.DMA,
            pltpu.SemaphoreType.DMA,
            pltpu.SemaphoreType.DMA,
            pltpu.SemaphoreType.DMA,
            pltpu.SemaphoreType.DMA,
            pltpu.SemaphoreType.DMA,
        ],
    )(ids_grouped, token_embedding, position_embedding)
    return out


@functools.lru_cache(maxsize=8)
def _jitted_run(sharding):
    if sharding is None:
        return jax.jit(_run)
    fmt = jex_layout.Format(
        jex_layout.Layout(major_to_minor=(0, 1, 2), tiling=()), sharding)
    return jax.jit(_run, out_shardings=fmt)


def kernel(input_ids, token_embedding, position_embedding):
    ids_grouped = input_ids.reshape(NUM_WORKERS, CHUNKS_PER_W, CHUNK)
    run = _jitted_run(getattr(token_embedding, "sharding", None))
    return run(ids_grouped, token_embedding, position_embedding)


# speedup vs baseline: 1.8978x; 1.2436x over previous
import functools

import jax
import jax.numpy as jnp
from jax import lax
from jax.experimental import pallas as pl
from jax.experimental.pallas import tpu as pltpu
from jax.experimental.pallas import tpu_sc as plsc

VOCAB = 49408
EMBED = 768
MAX_POS = 77
BATCH = 4096
SEQ = 77

NUM_WORKERS = 32
SEQS_PER_W = BATCH // NUM_WORKERS
PAIRS_PER_W = SEQS_PER_W // 2
POS_BLK = 8
MAIN_BLKS = 9
TAIL_ROWS = SEQ - MAIN_BLKS * POS_BLK
ROWS_PER_CHUNK = 2 * POS_BLK
MAIN_CHUNKS = PAIRS_PER_W * MAIN_BLKS
TAIL_CHUNKS = PAIRS_PER_W
CHUNKS_PER_W = MAIN_CHUNKS + TAIL_CHUNKS
IDX_PER_W = CHUNKS_PER_W * ROWS_PER_CHUNK
LANES = 16
VREGS_PER_ROW = EMBED // LANES
NBUF = 4


def _body(ids_hbm, tok_hbm, pos_hbm, out_hbm, idx_v, pos_v,
          buf0, buf1, buf2, buf3, g0, g1, g2, g3, o0, o1, o2, o3):
    wid = lax.axis_index("s") * 2 + lax.axis_index("c")
    seq0 = wid * SEQS_PER_W

    pltpu.sync_copy(ids_hbm.at[pl.ds(wid * IDX_PER_W, IDX_PER_W)], idx_v)
    pltpu.sync_copy(pos_hbm, pos_v)

    bufs = ((buf0, g0, o0), (buf1, g1, o1), (buf2, g2, o2), (buf3, g3, o3))

    def start_gather(c, buf, gsem):
        pltpu.async_copy(
            tok_hbm.at[idx_v.at[pl.ds(c * ROWS_PER_CHUNK, ROWS_PER_CHUNK)]],
            buf, gsem)

    def wait_gather(c, buf, gsem):
        pltpu.make_async_copy(
            tok_hbm.at[idx_v.at[pl.ds(c * ROWS_PER_CHUNK, ROWS_PER_CHUNK)]],
            buf, gsem).wait()

    def add_pos(buf, p0, nrows):
        def row_add(j, carry):
            prow = p0 + j

            def one_vreg(v):
                sl = pl.ds(v * LANES, LANES)
                pv = pos_v[prow, sl]
                plsc.addupdate(buf.at[j, sl], pv)
                plsc.addupdate(buf.at[POS_BLK + j, sl], pv)

            plsc.parallel_loop(0, VREGS_PER_ROW, unroll=8)(one_vreg)
            return carry

        lax.fori_loop(0, nrows, row_add, 0, unroll=False)

    def writeback(buf, osem, seq_a, p0, nrows):
        pltpu.async_copy(buf.at[pl.ds(0, nrows)],
                         out_hbm.at[seq_a, pl.ds(p0, nrows)], osem)
        pltpu.async_copy(buf.at[pl.ds(POS_BLK, nrows)],
                         out_hbm.at[seq_a + 1, pl.ds(p0, nrows)], osem)

    def drain_wb(buf, osem, nrows):
        pltpu.make_async_copy(buf.at[pl.ds(0, nrows)],
                              out_hbm.at[0, pl.ds(0, nrows)], osem).wait()
        pltpu.make_async_copy(buf.at[pl.ds(POS_BLK, nrows)],
                              out_hbm.at[0, pl.ds(0, nrows)], osem).wait()

    start_gather(0, bufs[0][0], bufs[0][1])
    start_gather(1, bufs[1][0], bufs[1][1])

    def main_body(c2, carry):
        p, t = carry
        for k in range(NBUF):
            c = NBUF * c2 + k
            buf, gsem, osem = bufs[k]
            wait_gather(c, buf, gsem)
            p0 = t * POS_BLK
            add_pos(buf, p0, POS_BLK)
            writeback(buf, osem, seq0 + 2 * p, p0, POS_BLK)
            twrap = t == MAIN_BLKS - 1
            p = p + twrap.astype(jnp.int32)
            t = lax.select(twrap, jnp.int32(0), t + 1)

            nbuf_, ngsem, nosem = bufs[(k + 2) % NBUF]

            @pl.when(c + 2 < MAIN_CHUNKS)
            def _():
                def drain_and_fetch():
                    drain_wb(nbuf_, nosem, POS_BLK)
                    start_gather(c + 2, nbuf_, ngsem)

                if k < 2:
                    @pl.when(c2 > 0)
                    def _():
                        drain_and_fetch()

                    @pl.when(c2 == 0)
                    def _():
                        start_gather(c + 2, nbuf_, ngsem)
                else:
                    drain_and_fetch()

        return (p, t)

    lax.fori_loop(0, MAIN_CHUNKS // NBUF, main_body,
                  (jnp.int32(0), jnp.int32(0)), unroll=False)

    for k in range(NBUF):
        buf, _, osem = bufs[k]
        drain_wb(buf, osem, POS_BLK)

    start_gather(MAIN_CHUNKS + 0, bufs[0][0], bufs[0][1])
    start_gather(MAIN_CHUNKS + 1, bufs[1][0], bufs[1][1])

    def tail_body(c2, carry):
        for k in range(NBUF):
            i = NBUF * c2 + k
            c = MAIN_CHUNKS + i
            buf, gsem, osem = bufs[k]
            wait_gather(c, buf, gsem)
            add_pos(buf, MAIN_BLKS * POS_BLK, TAIL_ROWS)
            writeback(buf, osem, seq0 + 2 * i, MAIN_BLKS * POS_BLK, TAIL_ROWS)

            nbuf_, ngsem, nosem = bufs[(k + 2) % NBUF]

            @pl.when(i + 2 < TAIL_CHUNKS)
            def _():
                def drain_and_fetch():
                    drain_wb(nbuf_, nosem, TAIL_ROWS)
                    start_gather(c + 2, nbuf_, ngsem)

                if k < 2:
                    @pl.when(c2 > 0)
                    def _():
                        drain_and_fetch()

                    @pl.when(c2 == 0)
                    def _():
                        start_gather(c + 2, nbuf_, ngsem)
                else:
                    drain_and_fetch()

        return carry

    lax.fori_loop(0, TAIL_CHUNKS // NBUF, tail_body, 0, unroll=False)

    for k in range(NBUF):
        buf, _, osem = bufs[k]
        drain_wb(buf, osem, TAIL_ROWS)


@jax.jit
def _run(ids_grouped, token_embedding, position_embedding):
    mesh = plsc.VectorSubcoreMesh(core_axis_name="c", subcore_axis_name="s")
    out = pl.kernel(
        _body,
        out_type=jax.ShapeDtypeStruct((BATCH, SEQ, EMBED), jnp.float32),
        mesh=mesh,
        scratch_types=[
            pltpu.VMEM((NUM_WORKERS * IDX_PER_W // NUM_WORKERS,), jnp.int32),
            pltpu.VMEM((MAX_POS, EMBED), jnp.float32),
            pltpu.VMEM((ROWS_PER_CHUNK, EMBED), jnp.float32),
            pltpu.VMEM((ROWS_PER_CHUNK, EMBED), jnp.float32),
            pltpu.VMEM((ROWS_PER_CHUNK, EMBED), jnp.float32),
            pltpu.VMEM((ROWS_PER_CHUNK, EMBED), jnp.float32),
            pltpu.SemaphoreType.DMA,
            pltpu.SemaphoreType.DMA,
            pltpu.SemaphoreType.DMA,
            pltpu.SemaphoreType.DMA,
            pltpu.SemaphoreType.DMA,
            pltpu.SemaphoreType.DMA,
            pltpu.SemaphoreType.DMA,
            pltpu.SemaphoreType.DMA,
        ],
    )(ids_grouped, token_embedding, position_embedding)
    return out


def _arrange_ids(input_ids):
    ids_pad = jnp.pad(input_ids, ((0, 0), (0, 80 - SEQ)))
    g = ids_pad.reshape(NUM_WORKERS, PAIRS_PER_W, 2, 10, POS_BLK)
    g = g.transpose(0, 1, 3, 2, 4)
    main = g[:, :, :MAIN_BLKS]
    tail = g[:, :, MAIN_BLKS:]
    main = main.reshape(NUM_WORKERS, MAIN_CHUNKS * ROWS_PER_CHUNK)
    tail = tail.reshape(NUM_WORKERS, TAIL_CHUNKS * ROWS_PER_CHUNK)
    return jnp.concatenate([main, tail], axis=1).reshape(-1)


def kernel(input_ids, token_embedding, position_embedding):
    ids_grouped = _arrange_ids(input_ids)
    return _run(ids_grouped, token_embedding, position_embedding)


# speedup vs baseline: 1.8994x; 1.0009x over previous
import jax
import jax.numpy as jnp
from jax import lax
from jax.experimental import pallas as pl
from jax.experimental.pallas import tpu as pltpu
from jax.experimental.pallas import tpu_sc as plsc

VOCAB = 49408
EMBED = 768
MAX_POS = 77
BATCH = 4096
SEQ = 77

NUM_WORKERS = 32
SEQS_PER_W = BATCH // NUM_WORKERS
PAIRS_PER_W = SEQS_PER_W // 2
POS_BLK = 8
MAIN_BLKS = 9
TAIL_ROWS = SEQ - MAIN_BLKS * POS_BLK
ROWS_PER_CHUNK = 2 * POS_BLK
MAIN_CHUNKS = PAIRS_PER_W * MAIN_BLKS
TAIL_CHUNKS = PAIRS_PER_W
CHUNKS_PER_W = MAIN_CHUNKS + TAIL_CHUNKS
IDX_PER_W = CHUNKS_PER_W * ROWS_PER_CHUNK
LANES = 16
VREGS_PER_ROW = EMBED // LANES
NBUF = 4


def _body(ids_hbm, tok_hbm, pos_hbm, out_hbm, idx_v, pos_v,
          buf0, buf1, buf2, buf3, g0, g1, g2, g3, o0, o1, o2, o3):
    wid = lax.axis_index("s") * 2 + lax.axis_index("c")
    seq0 = wid * SEQS_PER_W

    pltpu.sync_copy(ids_hbm.at[pl.ds(wid * IDX_PER_W, IDX_PER_W)], idx_v)
    pltpu.sync_copy(pos_hbm, pos_v)

    bufs = ((buf0, g0, o0), (buf1, g1, o1), (buf2, g2, o2), (buf3, g3, o3))

    def start_gather(c, buf, gsem):
        pltpu.async_copy(
            tok_hbm.at[idx_v.at[pl.ds(c * ROWS_PER_CHUNK, ROWS_PER_CHUNK)]],
            buf, gsem)

    def wait_gather(c, buf, gsem):
        pltpu.make_async_copy(
            tok_hbm.at[idx_v.at[pl.ds(c * ROWS_PER_CHUNK, ROWS_PER_CHUNK)]],
            buf, gsem).wait()

    def add_pos(buf, p0, nrows):
        def row_add(j, carry):
            prow = p0 + j

            def one_vreg(v):
                sl = pl.ds(v * LANES, LANES)
                pv = pos_v[prow, sl]
                plsc.addupdate(buf.at[j, sl], pv)
                plsc.addupdate(buf.at[POS_BLK + j, sl], pv)

            plsc.parallel_loop(0, VREGS_PER_ROW, unroll=8)(one_vreg)
            return carry

        lax.fori_loop(0, nrows, row_add, 0, unroll=False)

    def writeback(buf, osem, seq_a, p0, nrows):
        pltpu.async_copy(buf.at[pl.ds(0, nrows)],
                         out_hbm.at[seq_a, pl.ds(p0, nrows)], osem)
        pltpu.async_copy(buf.at[pl.ds(POS_BLK, nrows)],
                         out_hbm.at[seq_a + 1, pl.ds(p0, nrows)], osem)

    def drain_wb(buf, osem, nrows):
        pltpu.make_async_copy(buf.at[pl.ds(0, nrows)],
                              out_hbm.at[0, pl.ds(0, nrows)], osem).wait()
        pltpu.make_async_copy(buf.at[pl.ds(POS_BLK, nrows)],
                              out_hbm.at[0, pl.ds(0, nrows)], osem).wait()

    start_gather(0, bufs[0][0], bufs[0][1])
    start_gather(1, bufs[1][0], bufs[1][1])

    def main_body(c2, carry):
        p, t = carry
        for k in range(NBUF):
            c = NBUF * c2 + k
            buf, gsem, osem = bufs[k]
            wait_gather(c, buf, gsem)
            p0 = t * POS_BLK
            add_pos(buf, p0, POS_BLK)
            writeback(buf, osem, seq0 + 2 * p, p0, POS_BLK)
            twrap = t == MAIN_BLKS - 1
            p = p + twrap.astype(jnp.int32)
            t = lax.select(twrap, jnp.int32(0), t + 1)

            nbuf_, ngsem, nosem = bufs[(k + 2) % NBUF]

            @pl.when(c + 2 < MAIN_CHUNKS)
            def _():
                def drain_and_fetch():
                    drain_wb(nbuf_, nosem, POS_BLK)
                    start_gather(c + 2, nbuf_, ngsem)

                if k < 2:
                    @pl.when(c2 > 0)
                    def _():
                        drain_and_fetch()

                    @pl.when(c2 == 0)
                    def _():
                        start_gather(c + 2, nbuf_, ngsem)
                else:
                    drain_and_fetch()

        return (p, t)

    lax.fori_loop(0, MAIN_CHUNKS // NBUF, main_body,
                  (jnp.int32(0), jnp.int32(0)), unroll=False)

    for k in range(NBUF):
        buf, _, osem = bufs[k]
        drain_wb(buf, osem, POS_BLK)

    start_gather(MAIN_CHUNKS + 0, bufs[0][0], bufs[0][1])
    start_gather(MAIN_CHUNKS + 1, bufs[1][0], bufs[1][1])

    def tail_body(c2, carry):
        for k in range(NBUF):
            i = NBUF * c2 + k
            c = MAIN_CHUNKS + i
            buf, gsem, osem = bufs[k]
            wait_gather(c, buf, gsem)
            add_pos(buf, MAIN_BLKS * POS_BLK, TAIL_ROWS)
            writeback(buf, osem, seq0 + 2 * i, MAIN_BLKS * POS_BLK, TAIL_ROWS)

            nbuf_, ngsem, nosem = bufs[(k + 2) % NBUF]

            @pl.when(i + 2 < TAIL_CHUNKS)
            def _():
                def drain_and_fetch():
                    drain_wb(nbuf_, nosem, TAIL_ROWS)
                    start_gather(c + 2, nbuf_, ngsem)

                if k < 2:
                    @pl.when(c2 > 0)
                    def _():
                        drain_and_fetch()

                    @pl.when(c2 == 0)
                    def _():
                        start_gather(c + 2, nbuf_, ngsem)
                else:
                    drain_and_fetch()

        return carry

    lax.fori_loop(0, TAIL_CHUNKS // NBUF, tail_body, 0, unroll=False)

    for k in range(NBUF):
        buf, _, osem = bufs[k]
        drain_wb(buf, osem, TAIL_ROWS)


@jax.jit
def _run(ids_grouped, token_embedding, position_embedding):
    mesh = plsc.VectorSubcoreMesh(core_axis_name="c", subcore_axis_name="s")
    out = pl.kernel(
        _body,
        out_type=jax.ShapeDtypeStruct((BATCH, SEQ, EMBED), jnp.float32),
        mesh=mesh,
        scratch_types=[
            pltpu.VMEM((IDX_PER_W,), jnp.int32),
            pltpu.VMEM((MAX_POS, EMBED), jnp.float32),
            pltpu.VMEM((ROWS_PER_CHUNK, EMBED), jnp.float32),
            pltpu.VMEM((ROWS_PER_CHUNK, EMBED), jnp.float32),
            pltpu.VMEM((ROWS_PER_CHUNK, EMBED), jnp.float32),
            pltpu.VMEM((ROWS_PER_CHUNK, EMBED), jnp.float32),
            pltpu.SemaphoreType.DMA,
            pltpu.SemaphoreType.DMA,
            pltpu.SemaphoreType.DMA,
            pltpu.SemaphoreType.DMA,
            pltpu.SemaphoreType.DMA,
            pltpu.SemaphoreType.DMA,
            pltpu.SemaphoreType.DMA,
            pltpu.SemaphoreType.DMA,
        ],
    )(ids_grouped, token_embedding, position_embedding)
    return out


def _arrange_ids(input_ids):
    ids_pad = jnp.pad(input_ids, ((0, 0), (0, 80 - SEQ)))
    g = ids_pad.reshape(NUM_WORKERS, PAIRS_PER_W, 2, 10, POS_BLK)
    g = g.transpose(0, 1, 3, 2, 4)
    main = g[:, :, :MAIN_BLKS]
    tail = g[:, :, MAIN_BLKS:]
    main = main.reshape(NUM_WORKERS, MAIN_CHUNKS * ROWS_PER_CHUNK)
    tail = tail.reshape(NUM_WORKERS, TAIL_CHUNKS * ROWS_PER_CHUNK)
    return jnp.concatenate([main, tail], axis=1).reshape(-1)


def kernel(input_ids, token_embedding, position_embedding):
    ids_grouped = _arrange_ids(input_ids)
    return _run(ids_grouped, token_embedding, position_embedding)
